# baseline (speedup 1.0000x reference)
import functools
import jax
import jax.numpy as jnp
from jax import lax
from jax.experimental import pallas as pl
from jax.experimental.pallas import tpu as pltpu
from jax.experimental.pallas import tpu_sc as plsc

D_MODEL = 64
SCALE = 8.0

NC = 2
NS = 16
NW = NC * NS

N_T = 200
CHUNK = 128
N_B1 = 4096 // CHUNK

NI = 4
NO = 4


V = 1000000
N_FULL = V // CHUNK
TAIL = V - N_FULL * CHUNK
P1_ITERS = (N_FULL + NW - 1) // NW
P1_PAD = 130


def _prep_body(wt_hbm, tail_hbm, out_hbm, in_v, out_v, gsem, osem):
    w = lax.axis_index("s") * NC + lax.axis_index("c")
    iota = lax.iota(jnp.int32, 16)
    rowpair = iota // 2
    colpair = (iota % 2) * D_MODEL

    def gather(blk, bi):
        pltpu.async_copy(
            wt_hbm.at[:, pl.ds(CHUNK * blk, CHUNK)],
            in_v.at[pl.ds(bi * D_MODEL, D_MODEL), pl.ds(0, CHUNK)],
            gsem.at[bi],
        )

    def wait_gather(bi):
        pltpu.make_async_copy(
            wt_hbm.at[:, pl.ds(0, CHUNK)],
            in_v.at[pl.ds(0, D_MODEL), pl.ds(0, CHUNK)],
            gsem.at[bi],
        ).wait()

    def writeback(blk, bo):
        pltpu.async_copy(
            out_v.at[pl.ds(bo * D_MODEL, D_MODEL), pl.ds(0, 2 * D_MODEL)],
            out_hbm.at[pl.ds(D_MODEL * blk, D_MODEL)],
            osem.at[bo],
        )

    def wait_writeback(bo):
        pltpu.make_async_copy(
            out_v.at[pl.ds(0, D_MODEL), pl.ds(0, 2 * D_MODEL)],
            out_hbm.at[pl.ds(0, D_MODEL)],
            osem.at[bo],
        ).wait()

    def blk_of(i):
        return w + NW * i

    for b in range(NI):

        @pl.when(blk_of(b) < N_FULL)
        def _():
            gather(blk_of(b), b)

    def transpose_block(bi, bo, n_cg):
        in_base = bi * D_MODEL
        out_base = bo * D_MODEL

        @plsc.parallel_loop(0, n_cg, step=1, unroll=2)
        def do_cg(cg):
            rvec = rowpair + (out_base + 8 * cg)
            vals = [
                in_v[in_base + f, pl.ds(16 * cg, 16)] * SCALE
                for f in range(D_MODEL)
            ]
            for f in range(D_MODEL):
                plsc.store_scatter(out_v, [rvec, colpair + f], vals[f])

    def step(i, _):
        blk = blk_of(i)

        @pl.when(blk < N_FULL)
        def _():
            bi = lax.rem(i, NI)
            bo = lax.rem(i, NO)
            wait_gather(bi)

            @pl.when(i >= NO)
            def _():
                wait_writeback(bo)

            transpose_block(bi, bo, CHUNK // 16)
            writeback(blk, bo)

            nxt = blk_of(i + NI)

            @pl.when(nxt < N_FULL)
            def _():
                gather(nxt, bi)

        return ()

    lax.fori_loop(0, P1_ITERS, step, ())

    for i in range(P1_ITERS - NO, P1_ITERS):

        @pl.when(blk_of(i) < N_FULL)
        def _():
            wait_writeback(lax.rem(i, NO))

    @pl.when(w == NW - 1)
    def _():
        pltpu.sync_copy(tail_hbm, in_v.at[pl.ds(0, D_MODEL), pl.ds(0, CHUNK)])
        transpose_block(0, 0, CHUNK // 16)
        pltpu.sync_copy(
            out_v.at[pl.ds(0, D_MODEL), pl.ds(0, 2 * D_MODEL)],
            out_hbm.at[pl.ds(V // 2 - D_MODEL, D_MODEL)],
        )


def _emb_body(idx_hbm, table_hbm, out_hbm, idx_v, in_v, out_v, gsem, osem):
    w = lax.axis_index("s") * NC + lax.axis_index("c")

    pltpu.sync_copy(idx_hbm.at[w], idx_v)

    def gather(t, bi):
        return pltpu.async_copy(
            table_hbm.at[idx_v.at[t]],
            in_v.at[pl.ds(bi * CHUNK, CHUNK)],
            gsem.at[bi],
        )

    def wait_gather(t, bi):
        pltpu.make_async_copy(
            table_hbm.at[idx_v.at[t]],
            in_v.at[pl.ds(bi * CHUNK, CHUNK)],
            gsem.at[bi],
        ).wait()

    def writeback(t, bo):
        for ft in range(8):
            pltpu.async_copy(
                out_v.at[pl.ds(bo * D_MODEL + 8 * ft, 8), pl.ds(0, CHUNK)],
                out_hbm.at[t, ft, w],
                osem.at[bo],
            )

    def wait_writeback(bo):
        for ft in range(8):
            pltpu.make_async_copy(
                out_v.at[pl.ds(0, 8), pl.ds(0, CHUNK)],
                out_hbm.at[0, 0, w],
                osem.at[bo],
            ).wait()

    for b in range(NI):
        gather(b, b)

    iota = lax.iota(jnp.int32, 16)

    def step(t, _):
        bi = lax.rem(t, NI)
        bo = lax.rem(t, NO)
        wait_gather(t, bi)

        @pl.when(t >= NO)
        def _():
            wait_writeback(bo)

        in_base = bi * CHUNK
        out_base = bo * D_MODEL

        @plsc.parallel_loop(0, CHUNK, step=4, unroll=2)
        def do_row(c0):
            rows = [
                [in_v[in_base + c0 + cr, pl.ds(16 * k, 16)]
                 for k in range(D_MODEL // 16)]
                for cr in range(4)
            ]
            for cr in range(4):
                csplat = jnp.full((16,), c0 + cr, jnp.int32)
                for k in range(D_MODEL // 16):
                    plsc.store_scatter(
                        out_v, [out_base + 16 * k + iota, csplat], rows[cr][k]
                    )
        writeback(t, bo)

        @pl.when(t + NI < N_T)
        def _():
            gather(t + NI, bi)

        return ()

    lax.fori_loop(0, N_T, step, ())

    for t in range(N_T - NO, N_T):
        wait_writeback(t % NO)


@jax.jit
def _prep_call(wt, tail_wt):
    mesh = plsc.VectorSubcoreMesh(
        core_axis_name="c", subcore_axis_name="s", num_cores=NC, num_subcores=NS
    )
    fn = pl.kernel(
        _prep_body,
        out_type=jax.ShapeDtypeStruct((V // 2, 2 * D_MODEL), jnp.float32),
        mesh=mesh,
        scratch_types=[
            pltpu.VMEM((NI * D_MODEL, CHUNK), jnp.float32),
            pltpu.VMEM((NO * D_MODEL, P1_PAD), jnp.float32),
            pltpu.SemaphoreType.DMA((NI,)),
            pltpu.SemaphoreType.DMA((NO,)),
        ],
        compiler_params=pltpu.CompilerParams(
            use_tc_tiling_on_sc=True,
            needs_layout_passes=False,
            disable_bounds_checks=True,
        ),
    )
    return fn(wt, tail_wt)


@jax.jit
def _emb_call(idx, weight):
    mesh = plsc.VectorSubcoreMesh(
        core_axis_name="c", subcore_axis_name="s", num_cores=NC, num_subcores=NS
    )
    fn = pl.kernel(
        _emb_body,
        out_type=jax.ShapeDtypeStruct((N_T, 8, N_B1, 8, CHUNK), jnp.float32),
        mesh=mesh,
        scratch_types=[
            pltpu.VMEM((N_T, CHUNK), jnp.int32),
            pltpu.VMEM((NI * CHUNK, D_MODEL), jnp.float32),
            pltpu.VMEM((NO * D_MODEL, CHUNK + 1), jnp.float32),
            pltpu.SemaphoreType.DMA((NI,)),
            pltpu.SemaphoreType.DMA((NO,)),
        ],
        compiler_params=pltpu.CompilerParams(
            use_tc_tiling_on_sc=False,
            needs_layout_passes=False,
            disable_bounds_checks=True,
        ),
    )
    return fn(idx, weight)


def kernel(token_embedding, weight):
    idx = token_embedding.T.reshape(N_T, N_B1, CHUNK).transpose(1, 0, 2)
    table = _prep_call(weight.T, weight[V - CHUNK :, :].T).reshape(V, D_MODEL)
    out5 = _emb_call(idx, table)
    return out5.transpose(2, 4, 0, 1, 3).reshape(4096, N_T, D_MODEL)


# speedup vs baseline: 1.4297x; 1.4297x over previous
import functools
import jax
import jax.numpy as jnp
from jax import lax
from jax.experimental import pallas as pl
from jax.experimental.pallas import tpu as pltpu
from jax.experimental.pallas import tpu_sc as plsc

D_MODEL = 64
SCALE = 8.0

NC = 2
NS = 16
NW = NC * NS

N_T = 200
CHUNK = 128
N_B1 = 4096 // CHUNK

NI = 4
NO = 4


V = 1000000
N_FULL = V // CHUNK
TAIL = V - N_FULL * CHUNK
P1_ITERS = (N_FULL + NW - 1) // NW
P1_PAD = 130


def _prep_body(wt_hbm, tail_hbm, out_hbm, in_v, out_v, gsem, osem):
    w = lax.axis_index("s") * NC + lax.axis_index("c")
    iota = lax.iota(jnp.int32, 16)
    rots = [(iota + d) % 16 for d in range(16)]
    halfs = iota // 2

    def gather(blk, bi):
        pltpu.async_copy(
            wt_hbm.at[:, pl.ds(CHUNK * blk, CHUNK)],
            in_v.at[pl.ds(bi * D_MODEL, D_MODEL), pl.ds(0, CHUNK)],
            gsem.at[bi],
        )

    def wait_gather(bi):
        pltpu.make_async_copy(
            wt_hbm.at[:, pl.ds(0, CHUNK)],
            in_v.at[pl.ds(0, D_MODEL), pl.ds(0, CHUNK)],
            gsem.at[bi],
        ).wait()

    def writeback(blk, bo):
        pltpu.async_copy(
            out_v.at[pl.ds(bo * D_MODEL, D_MODEL), pl.ds(0, 2 * D_MODEL)],
            out_hbm.at[pl.ds(D_MODEL * blk, D_MODEL)],
            osem.at[bo],
        )

    def wait_writeback(bo):
        pltpu.make_async_copy(
            out_v.at[pl.ds(0, D_MODEL), pl.ds(0, 2 * D_MODEL)],
            out_hbm.at[pl.ds(0, D_MODEL)],
            osem.at[bo],
        ).wait()

    def blk_of(i):
        return w + NW * i

    for b in range(NI):

        @pl.when(blk_of(b) < N_FULL)
        def _():
            gather(blk_of(b), b)

    def transpose_block(bi, bo, n_cg):
        in_base = bi * D_MODEL
        out_base = bo * D_MODEL

        @plsc.parallel_loop(0, n_cg, step=1, unroll=2)
        def do_cg(cg):
            cvec = 16 * cg + iota
            prow = out_base + 8 * cg + halfs
            for f0 in range(0, D_MODEL, 16):
                for d in range(16):
                    vals = plsc.load_gather(
                        in_v, [in_base + f0 + rots[d], cvec]
                    )
                    plsc.store_scatter(
                        out_v,
                        [prow, (iota % 2) * D_MODEL + f0 + rots[d]],
                        vals * SCALE,
                    )

    def step(i, _):
        blk = blk_of(i)

        @pl.when(blk < N_FULL)
        def _():
            bi = lax.rem(i, NI)
            bo = lax.rem(i, NO)
            wait_gather(bi)

            @pl.when(i >= NO)
            def _():
                wait_writeback(bo)

            transpose_block(bi, bo, CHUNK // 16)
            writeback(blk, bo)

            nxt = blk_of(i + NI)

            @pl.when(nxt < N_FULL)
            def _():
                gather(nxt, bi)

        return ()

    lax.fori_loop(0, P1_ITERS, step, ())

    for i in range(P1_ITERS - NO, P1_ITERS):

        @pl.when(blk_of(i) < N_FULL)
        def _():
            wait_writeback(lax.rem(i, NO))

    @pl.when(w == NW - 1)
    def _():
        pltpu.sync_copy(tail_hbm, in_v.at[pl.ds(0, D_MODEL), pl.ds(0, CHUNK)])
        transpose_block(0, 0, CHUNK // 16)
        pltpu.sync_copy(
            out_v.at[pl.ds(0, D_MODEL), pl.ds(0, 2 * D_MODEL)],
            out_hbm.at[pl.ds(V // 2 - D_MODEL, D_MODEL)],
        )


def _emb_body(idx_hbm, table_hbm, out_hbm, idx_v, in_v, out_v, gsem, osem):
    w = lax.axis_index("s") * NC + lax.axis_index("c")

    pltpu.sync_copy(idx_hbm.at[w], idx_v)

    def gather(t, bi):
        return pltpu.async_copy(
            table_hbm.at[idx_v.at[t]],
            in_v.at[pl.ds(bi * CHUNK, CHUNK)],
            gsem.at[bi],
        )

    def wait_gather(t, bi):
        pltpu.make_async_copy(
            table_hbm.at[idx_v.at[t]],
            in_v.at[pl.ds(bi * CHUNK, CHUNK)],
            gsem.at[bi],
        ).wait()

    def writeback(t, bo):
        for ft in range(8):
            pltpu.async_copy(
                out_v.at[pl.ds(bo * D_MODEL + 8 * ft, 8), pl.ds(0, CHUNK)],
                out_hbm.at[t, ft, w],
                osem.at[bo],
            )

    def wait_writeback(bo):
        for ft in range(8):
            pltpu.make_async_copy(
                out_v.at[pl.ds(0, 8), pl.ds(0, CHUNK)],
                out_hbm.at[0, 0, w],
                osem.at[bo],
            ).wait()

    for b in range(NI):
        gather(b, b)

    iota = lax.iota(jnp.int32, 16)

    def step(t, _):
        bi = lax.rem(t, NI)
        bo = lax.rem(t, NO)
        wait_gather(t, bi)

        @pl.when(t >= NO)
        def _():
            wait_writeback(bo)

        in_base = bi * CHUNK
        out_base = bo * D_MODEL

        @plsc.parallel_loop(0, CHUNK, step=4, unroll=2)
        def do_row(c0):
            rows = [
                [in_v[in_base + c0 + cr, pl.ds(16 * k, 16)]
                 for k in range(D_MODEL // 16)]
                for cr in range(4)
            ]
            for cr in range(4):
                csplat = jnp.full((16,), c0 + cr, jnp.int32)
                for k in range(D_MODEL // 16):
                    plsc.store_scatter(
                        out_v, [out_base + 16 * k + iota, csplat], rows[cr][k]
                    )
        writeback(t, bo)

        @pl.when(t + NI < N_T)
        def _():
            gather(t + NI, bi)

        return ()

    lax.fori_loop(0, N_T, step, ())

    for t in range(N_T - NO, N_T):
        wait_writeback(t % NO)


@jax.jit
def _prep_call(wt, tail_wt):
    mesh = plsc.VectorSubcoreMesh(
        core_axis_name="c", subcore_axis_name="s", num_cores=NC, num_subcores=NS
    )
    fn = pl.kernel(
        _prep_body,
        out_type=jax.ShapeDtypeStruct((V // 2, 2 * D_MODEL), jnp.float32),
        mesh=mesh,
        scratch_types=[
            pltpu.VMEM((NI * D_MODEL, CHUNK), jnp.float32),
            pltpu.VMEM((NO * D_MODEL, 2 * D_MODEL), jnp.float32),
            pltpu.SemaphoreType.DMA((NI,)),
            pltpu.SemaphoreType.DMA((NO,)),
        ],
        compiler_params=pltpu.CompilerParams(
            use_tc_tiling_on_sc=True,
            needs_layout_passes=False,
            disable_bounds_checks=True,
        ),
    )
    return fn(wt, tail_wt)


@jax.jit
def _emb_call(idx, weight):
    mesh = plsc.VectorSubcoreMesh(
        core_axis_name="c", subcore_axis_name="s", num_cores=NC, num_subcores=NS
    )
    fn = pl.kernel(
        _emb_body,
        out_type=jax.ShapeDtypeStruct((N_T, 8, N_B1, 8, CHUNK), jnp.float32),
        mesh=mesh,
        scratch_types=[
            pltpu.VMEM((N_T, CHUNK), jnp.int32),
            pltpu.VMEM((NI * CHUNK, D_MODEL), jnp.float32),
            pltpu.VMEM((NO * D_MODEL, CHUNK + 1), jnp.float32),
            pltpu.SemaphoreType.DMA((NI,)),
            pltpu.SemaphoreType.DMA((NO,)),
        ],
        compiler_params=pltpu.CompilerParams(
            use_tc_tiling_on_sc=False,
            needs_layout_passes=False,
            disable_bounds_checks=True,
        ),
    )
    return fn(idx, weight)


def kernel(token_embedding, weight):
    idx = token_embedding.T.reshape(N_T, N_B1, CHUNK).transpose(1, 0, 2)
    table = _prep_call(weight.T, weight[V - CHUNK :, :].T).reshape(V, D_MODEL)
    out5 = _emb_call(idx, table)
    return out5.transpose(2, 4, 0, 1, 3).reshape(4096, N_T, D_MODEL)


# speedup vs baseline: 1.7377x; 1.2154x over previous
import functools
import jax
import jax.numpy as jnp
from jax import lax
from jax.experimental import pallas as pl
from jax.experimental.pallas import tpu as pltpu
from jax.experimental.pallas import tpu_sc as plsc

D_MODEL = 64
SCALE = 8.0

NC = 2
NS = 16
NW = NC * NS

N_T = 200
CHUNK = 128
N_B1 = 4096 // CHUNK

NI = 4
NO = 4


def _emb_body(idx_hbm, table_hbm, out_hbm, idx_v, in_v, out_v, gsem, osem):
    w = lax.axis_index("s") * NC + lax.axis_index("c")

    pltpu.sync_copy(idx_hbm.at[w], idx_v)

    def gather(t, bi):
        return pltpu.async_copy(
            table_hbm.at[idx_v.at[t]],
            in_v.at[pl.ds(bi * CHUNK, CHUNK)],
            gsem.at[bi],
        )

    def wait_gather(t, bi):
        pltpu.make_async_copy(
            table_hbm.at[idx_v.at[t]],
            in_v.at[pl.ds(bi * CHUNK, CHUNK)],
            gsem.at[bi],
        ).wait()

    def writeback(t, bo):
        for ft in range(8):
            pltpu.async_copy(
                out_v.at[pl.ds(bo * D_MODEL + 8 * ft, 8), pl.ds(0, CHUNK)],
                out_hbm.at[t, ft, w],
                osem.at[bo],
            )

    def wait_writeback(bo):
        for ft in range(8):
            pltpu.make_async_copy(
                out_v.at[pl.ds(0, 8), pl.ds(0, CHUNK)],
                out_hbm.at[0, 0, w],
                osem.at[bo],
            ).wait()

    for b in range(NI):
        gather(b, b)

    iota = lax.iota(jnp.int32, 16)

    def step(t, _):
        bi = lax.rem(t, NI)
        bo = lax.rem(t, NO)
        wait_gather(t, bi)

        @pl.when(t >= NO)
        def _():
            wait_writeback(bo)

        in_base = bi * CHUNK
        out_base = bo * D_MODEL

        @plsc.parallel_loop(0, CHUNK, step=4, unroll=2)
        def do_row(c0):
            rows = [
                [in_v[in_base + c0 + cr, pl.ds(16 * k, 16)] * SCALE
                 for k in range(D_MODEL // 16)]
                for cr in range(4)
            ]
            for cr in range(4):
                csplat = jnp.full((16,), c0 + cr, jnp.int32)
                for k in range(D_MODEL // 16):
                    plsc.store_scatter(
                        out_v, [out_base + 16 * k + iota, csplat], rows[cr][k]
                    )
        writeback(t, bo)

        @pl.when(t + NI < N_T)
        def _():
            gather(t + NI, bi)

        return ()

    lax.fori_loop(0, N_T, step, ())

    for t in range(N_T - NO, N_T):
        wait_writeback(t % NO)


@jax.jit
def _emb_call(idx, weight):
    mesh = plsc.VectorSubcoreMesh(
        core_axis_name="c", subcore_axis_name="s", num_cores=NC, num_subcores=NS
    )
    fn = pl.kernel(
        _emb_body,
        out_type=jax.ShapeDtypeStruct((N_T, 8, N_B1, 8, CHUNK), jnp.float32),
        mesh=mesh,
        scratch_types=[
            pltpu.VMEM((N_T, CHUNK), jnp.int32),
            pltpu.VMEM((NI * CHUNK, D_MODEL), jnp.float32),
            pltpu.VMEM((NO * D_MODEL, CHUNK + 1), jnp.float32),
            pltpu.SemaphoreType.DMA((NI,)),
            pltpu.SemaphoreType.DMA((NO,)),
        ],
        compiler_params=pltpu.CompilerParams(
            use_tc_tiling_on_sc=False,
            needs_layout_passes=False,
            disable_bounds_checks=True,
        ),
    )
    return fn(idx, weight)


def kernel(token_embedding, weight):
    idx = token_embedding.T.reshape(N_T, N_B1, CHUNK).transpose(1, 0, 2)
    out5 = _emb_call(idx, weight)
    return out5.transpose(2, 4, 0, 1, 3).reshape(4096, N_T, D_MODEL)
